# Initial kernel scaffold; baseline (speedup 1.0000x reference)
import jax
import jax.numpy as jnp
from jax.experimental import pallas as pl


def kernel(x, edge_index, edge_weight, Wz, bz, Wlz, blz, Wr, br, Wlr, blr, Wh, bh, Wlh, blh, Wlin, blin):
    raise NotImplementedError("write your pallas kernel here")


# speedup vs baseline: 18.9414x; 18.9414x over previous
import functools

import jax
import jax.numpy as jnp
from jax import lax
from jax.experimental import pallas as pl
from jax.experimental.pallas import tpu as pltpu
from jax.experimental.pallas import tpu_sc as plsc

_SEQ = 15
_NN = 3
_NE = 9
_FIN = 512
_HID = 32

_MCOL = _NE
_MROW = _MCOL + _NE * _NN
_MKEY = _MROW + _NE * _NN
_PACK = _MKEY + _NE * _NE


def _sc_norm_body(pack_hbm, a_hbm, pack_v, a_v):
    c = lax.axis_index("c")
    s = lax.axis_index("s")

    @pl.when((s == 0) & (c == 0))
    def _():
        pltpu.sync_copy(pack_hbm, pack_v)
        ew = [pack_v[e, :] for e in range(_NE)]

        deg = []
        for n in range(_NN):
            d = jnp.full((16,), 1.0, jnp.float32)
            for e in range(_NE):
                d = d + pack_v[_MCOL + _NN * e + n, :] * ew[e]
            deg.append(d)

        dinv = []
        for n in range(_NN):
            x = deg[n]
            y = 1.0 / x
            for _ in range(12):
                y = y * (1.5 - (0.5 * x) * y * y)
            dinv.append(y)

        a = [jnp.zeros((16,), jnp.float32)] * _NE
        for n in range(_NN):
            a[4 * n] = dinv[n] * dinv[n]
        for e in range(_NE):
            dr = jnp.zeros((16,), jnp.float32)
            dc = jnp.zeros((16,), jnp.float32)
            for n in range(_NN):
                dr = dr + pack_v[_MROW + _NN * e + n, :] * dinv[n]
                dc = dc + pack_v[_MCOL + _NN * e + n, :] * dinv[n]
            norm_e = ew[e] * dr * dc
            for j in range(_NE):
                a[j] = a[j] + pack_v[_MKEY + _NE * e + j, :] * norm_e
        for j in range(_NE):
            a_v[j, :] = a[j]
        pltpu.sync_copy(a_v, a_hbm)


@functools.cache
def _get_sc_norm():
    return functools.partial(
        pl.kernel,
        out_type=jax.ShapeDtypeStruct((_NE, 16), jnp.float32),
        mesh=plsc.VectorSubcoreMesh(core_axis_name="c", subcore_axis_name="s"),
        scratch_types=[
            pltpu.VMEM((_PACK, 16), jnp.float32),
            pltpu.VMEM((_NE, 16), jnp.float32),
        ],
    )(_sc_norm_body)


def _tc_body(xn_ref, wz_ref, wh_ref, wlz_ref, wlh_ref, bz_ref, bh_ref,
             blz_ref, blh_ref, a_ref, wlin_ref, blin_ref, out_ref):
    X = xn_ref[...]
    W = jnp.concatenate([wz_ref[...], wh_ref[...]], axis=1)
    XW = jnp.dot(X, W, preferred_element_type=jnp.float32)
    A = a_ref[...]

    parts = []
    for cn in range(_NN):
        acc = (A[:, 3 * cn + 0:3 * cn + 1] * XW[0:15, :]
               + A[:, 3 * cn + 1:3 * cn + 2] * XW[15:30, :]
               + A[:, 3 * cn + 2:3 * cn + 3] * XW[30:45, :])
        parts.append(acc)
    G = jnp.concatenate(parts, axis=0)

    Gz = G[:, 0:_HID] + bz_ref[...]
    Gh = G[:, _HID:2 * _HID] + bh_ref[...]
    Z = jax.nn.sigmoid(
        jnp.dot(Gz, wlz_ref[0:_HID, :], preferred_element_type=jnp.float32)
        + blz_ref[...])
    Ht = jnp.tanh(
        jnp.dot(Gh, wlh_ref[0:_HID, :], preferred_element_type=jnp.float32)
        + blh_ref[...])
    h = jnp.maximum((1.0 - Z) * Ht, 0.0)

    o = blin_ref[...]
    for n in range(_NN):
        pn = jnp.sum(h[_SEQ * n:_SEQ * (n + 1), :], axis=0, keepdims=True) \
            * (1.0 / _SEQ)
        o = o + jnp.dot(pn, wlin_ref[_HID * n:_HID * (n + 1), :],
                        preferred_element_type=jnp.float32)
    out_ref[...] = o


_tc_core = pl.pallas_call(
    _tc_body,
    out_shape=jax.ShapeDtypeStruct((1, 2), jnp.float32),
)


def kernel(x, edge_index, edge_weight, Wz, bz, Wlz, blz, Wr, br, Wlr, blr,
           Wh, bh, Wlh, blh, Wlin, blin):
    row = edge_index[0].astype(jnp.int32)
    col = edge_index[1].astype(jnp.int32)
    n3 = jnp.arange(_NN, dtype=jnp.int32)
    n9 = jnp.arange(_NE, dtype=jnp.int32)
    mcol = (col[:, None] == n3).astype(jnp.float32).reshape(-1)
    mrow = (row[:, None] == n3).astype(jnp.float32).reshape(-1)
    mkey = ((3 * col + row)[:, None] == n9).astype(jnp.float32).reshape(-1)
    masks = jnp.concatenate([mcol, mrow, mkey])
    ew16 = jnp.zeros((_NE, 16), jnp.float32).at[:, :_SEQ].set(edge_weight.T)
    pack = jnp.concatenate(
        [ew16, jnp.broadcast_to(masks[:, None], (_PACK - _NE, 16))], axis=0)

    A_t = _get_sc_norm()(pack)
    A = A_t[:, :_SEQ].T

    Xn = x.transpose(1, 0, 2).reshape(_NN * _SEQ, _FIN)
    Wlin_nm = Wlin.reshape(_HID, _NN, -1).transpose(1, 0, 2).reshape(
        _NN * _HID, -1)

    out = _tc_core(Xn, Wz, Wh, Wlz, Wlh,
                   bz.reshape(1, _HID), bh.reshape(1, _HID),
                   blz.reshape(1, _HID), blh.reshape(1, _HID),
                   A, Wlin_nm, blin.reshape(1, -1))
    return out.reshape(-1)


# speedup vs baseline: 19.9467x; 1.0531x over previous
import functools

import jax
import jax.numpy as jnp
from jax import lax
from jax.experimental import pallas as pl
from jax.experimental.pallas import tpu as pltpu
from jax.experimental.pallas import tpu_sc as plsc

_SEQ = 15
_NN = 3
_NE = 9
_FIN = 512
_HID = 32

_MCOL = _NE
_MROW = _MCOL + _NE * _NN
_MKEY = _MROW + _NE * _NN
_PACK = _MKEY + _NE * _NE


def _sc_norm_body(pack_hbm, a_hbm, pack_v, a_v):
    c = lax.axis_index("c")
    s = lax.axis_index("s")

    @pl.when((s == 0) & (c == 0))
    def _():
        pltpu.sync_copy(pack_hbm, pack_v)
        ew = [pack_v[e, :] for e in range(_NE)]

        deg = []
        for n in range(_NN):
            d = jnp.full((16,), 1.0, jnp.float32)
            for e in range(_NE):
                d = d + pack_v[_MCOL + _NN * e + n, :] * ew[e]
            deg.append(d)

        dinv = []
        for n in range(_NN):
            x = deg[n]
            y = 1.0 / x
            for _ in range(12):
                y = y * (1.5 - (0.5 * x) * y * y)
            dinv.append(y)

        a = [jnp.zeros((16,), jnp.float32)] * _NE
        for n in range(_NN):
            a[4 * n] = dinv[n] * dinv[n]
        for e in range(_NE):
            dr = jnp.zeros((16,), jnp.float32)
            dc = jnp.zeros((16,), jnp.float32)
            for n in range(_NN):
                dr = dr + pack_v[_MROW + _NN * e + n, :] * dinv[n]
                dc = dc + pack_v[_MCOL + _NN * e + n, :] * dinv[n]
            norm_e = ew[e] * dr * dc
            for j in range(_NE):
                a[j] = a[j] + pack_v[_MKEY + _NE * e + j, :] * norm_e
        for j in range(_NE):
            a_v[j, :] = a[j]
        pltpu.sync_copy(a_v, a_hbm)


@functools.cache
def _get_sc_norm():
    return functools.partial(
        pl.kernel,
        out_type=jax.ShapeDtypeStruct((_NE, 16), jnp.float32),
        mesh=plsc.VectorSubcoreMesh(core_axis_name="c", subcore_axis_name="s"),
        scratch_types=[
            pltpu.VMEM((_PACK, 16), jnp.float32),
            pltpu.VMEM((_NE, 16), jnp.float32),
        ],
    )(_sc_norm_body)


def _tc_body(xn_ref, wz_ref, wh_ref, wlz_ref, wlh_ref, bz_ref, bh_ref,
             blz_ref, blh_ref, a_ref, wlin_ref, blin_ref, out_ref):
    X = xn_ref[...].transpose(1, 0, 2).reshape(_NN * _SEQ, _FIN)
    W = jnp.concatenate([wz_ref[...], wh_ref[...]], axis=1)
    XW = jnp.dot(X, W, preferred_element_type=jnp.float32)
    A = a_ref[...][:, :_SEQ].T

    parts = []
    for cn in range(_NN):
        acc = (A[:, 3 * cn + 0:3 * cn + 1] * XW[0:15, :]
               + A[:, 3 * cn + 1:3 * cn + 2] * XW[15:30, :]
               + A[:, 3 * cn + 2:3 * cn + 3] * XW[30:45, :])
        parts.append(acc)
    G = jnp.concatenate(parts, axis=0)

    Gz = G[:, 0:_HID] + bz_ref[...]
    Gh = G[:, _HID:2 * _HID] + bh_ref[...]
    Z = jax.nn.sigmoid(
        jnp.dot(Gz, wlz_ref[0:_HID, :], preferred_element_type=jnp.float32)
        + blz_ref[...])
    Ht = jnp.tanh(
        jnp.dot(Gh, wlh_ref[0:_HID, :], preferred_element_type=jnp.float32)
        + blh_ref[...])
    h = jnp.maximum((1.0 - Z) * Ht, 0.0)

    o = blin_ref[...]
    for n in range(_NN):
        pn = jnp.sum(h[_SEQ * n:_SEQ * (n + 1), :], axis=0, keepdims=True) \
            * (1.0 / _SEQ)
        o = o + jnp.dot(pn, wlin_ref[_HID * n:_HID * (n + 1), :],
                        preferred_element_type=jnp.float32)
    out_ref[...] = o


_tc_core = pl.pallas_call(
    _tc_body,
    out_shape=jax.ShapeDtypeStruct((1, 2), jnp.float32),
)


def kernel(x, edge_index, edge_weight, Wz, bz, Wlz, blz, Wr, br, Wlr, blr,
           Wh, bh, Wlh, blh, Wlin, blin):
    row = edge_index[0].astype(jnp.int32)
    col = edge_index[1].astype(jnp.int32)
    n3 = jnp.arange(_NN, dtype=jnp.int32)
    n9 = jnp.arange(_NE, dtype=jnp.int32)
    mcol = (col[:, None] == n3).astype(jnp.float32).reshape(-1)
    mrow = (row[:, None] == n3).astype(jnp.float32).reshape(-1)
    mkey = ((3 * col + row)[:, None] == n9).astype(jnp.float32).reshape(-1)
    masks = jnp.concatenate([mcol, mrow, mkey])
    ew16 = jnp.zeros((_NE, 16), jnp.float32).at[:, :_SEQ].set(edge_weight.T)
    pack = jnp.concatenate(
        [ew16, jnp.broadcast_to(masks[:, None], (_PACK - _NE, 16))], axis=0)

    A_t = _get_sc_norm()(pack)

    Wlin_nm = Wlin.reshape(_HID, _NN, -1).transpose(1, 0, 2).reshape(
        _NN * _HID, -1)

    out = _tc_core(x, Wz, Wh, Wlz, Wlh,
                   bz.reshape(1, _HID), bh.reshape(1, _HID),
                   blz.reshape(1, _HID), blh.reshape(1, _HID),
                   A_t, Wlin_nm, blin.reshape(1, -1))
    return out.reshape(-1)


# speedup vs baseline: 20.0569x; 1.0055x over previous
import functools

import jax
import jax.numpy as jnp
from jax import lax
from jax.experimental import pallas as pl
from jax.experimental.pallas import tpu as pltpu
from jax.experimental.pallas import tpu_sc as plsc

_SEQ = 15
_NN = 3
_NE = 9
_FIN = 512
_HID = 32

_MCOL = _NE
_MROW = _MCOL + _NE * _NN
_PACK = _MROW + _NE * _NN


def _sc_norm_body(pack_hbm, a_hbm, pack_v, a_v):
    c = lax.axis_index("c")
    s = lax.axis_index("s")

    @pl.when((s == 0) & (c == 0))
    def _():
        pltpu.sync_copy(pack_hbm, pack_v)
        ew = [pack_v[e, :] for e in range(_NE)]

        deg = []
        for n in range(_NN):
            d = jnp.full((16,), 1.0, jnp.float32)
            for e in range(_NE):
                d = d + pack_v[_MCOL + _NN * e + n, :] * ew[e]
            deg.append(d)

        dinv = []
        for n in range(_NN):
            x = deg[n]
            y = 1.0 / x
            for _ in range(12):
                y = y * (1.5 - (0.5 * x) * y * y)
            dinv.append(y)

        a = [jnp.zeros((16,), jnp.float32)] * _NE
        for n in range(_NN):
            a[4 * n] = dinv[n] * dinv[n]
        for e in range(_NE):
            mr = [pack_v[_MROW + _NN * e + n, :] for n in range(_NN)]
            mc = [pack_v[_MCOL + _NN * e + n, :] for n in range(_NN)]
            dr = mr[0] * dinv[0] + mr[1] * dinv[1] + mr[2] * dinv[2]
            dc = mc[0] * dinv[0] + mc[1] * dinv[1] + mc[2] * dinv[2]
            norm_e = ew[e] * dr * dc
            for cn in range(_NN):
                t = mc[cn] * norm_e
                for r in range(_NN):
                    a[3 * cn + r] = a[3 * cn + r] + mr[r] * t
        for j in range(_NE):
            a_v[j, :] = a[j]
        pltpu.sync_copy(a_v, a_hbm)


@functools.cache
def _get_sc_norm():
    return functools.partial(
        pl.kernel,
        out_type=jax.ShapeDtypeStruct((_NE, 16), jnp.float32),
        mesh=plsc.VectorSubcoreMesh(core_axis_name="c", subcore_axis_name="s"),
        scratch_types=[
            pltpu.VMEM((_PACK, 16), jnp.float32),
            pltpu.VMEM((_NE, 16), jnp.float32),
        ],
    )(_sc_norm_body)


def _tc_body(xn_ref, wz_ref, wh_ref, wlz_ref, wlh_ref, bz_ref, bh_ref,
             blz_ref, blh_ref, a_ref, wlin_ref, blin_ref, out_ref):
    X = xn_ref[...].transpose(1, 0, 2).reshape(_NN * _SEQ, _FIN)
    W = jnp.concatenate([wz_ref[...], wh_ref[...]], axis=1)
    XW = jnp.dot(X, W, preferred_element_type=jnp.float32)
    A = a_ref[...][:, :_SEQ].T

    parts = []
    for cn in range(_NN):
        acc = (A[:, 3 * cn + 0:3 * cn + 1] * XW[0:15, :]
               + A[:, 3 * cn + 1:3 * cn + 2] * XW[15:30, :]
               + A[:, 3 * cn + 2:3 * cn + 3] * XW[30:45, :])
        parts.append(acc)
    G = jnp.concatenate(parts, axis=0)

    Gz = G[:, 0:_HID] + bz_ref[...]
    Gh = G[:, _HID:2 * _HID] + bh_ref[...]
    Z = jax.nn.sigmoid(
        jnp.dot(Gz, wlz_ref[0:_HID, :], preferred_element_type=jnp.float32)
        + blz_ref[...])
    Ht = jnp.tanh(
        jnp.dot(Gh, wlh_ref[0:_HID, :], preferred_element_type=jnp.float32)
        + blh_ref[...])
    h = jnp.maximum((1.0 - Z) * Ht, 0.0)

    o = blin_ref[...]
    for n in range(_NN):
        pn = jnp.sum(h[_SEQ * n:_SEQ * (n + 1), :], axis=0, keepdims=True) \
            * (1.0 / _SEQ)
        o = o + jnp.dot(pn, wlin_ref[_HID * n:_HID * (n + 1), :],
                        preferred_element_type=jnp.float32)
    out_ref[...] = o


_tc_core = pl.pallas_call(
    _tc_body,
    out_shape=jax.ShapeDtypeStruct((1, 2), jnp.float32),
)


def kernel(x, edge_index, edge_weight, Wz, bz, Wlz, blz, Wr, br, Wlr, blr,
           Wh, bh, Wlh, blh, Wlin, blin):
    row = edge_index[0].astype(jnp.int32)
    col = edge_index[1].astype(jnp.int32)
    n3 = jnp.arange(_NN, dtype=jnp.int32)
    mcol = (col[:, None] == n3).astype(jnp.float32).reshape(-1)
    mrow = (row[:, None] == n3).astype(jnp.float32).reshape(-1)
    masks = jnp.concatenate([mcol, mrow])
    ew16 = jnp.zeros((_NE, 16), jnp.float32).at[:, :_SEQ].set(edge_weight.T)
    pack = jnp.concatenate(
        [ew16, jnp.broadcast_to(masks[:, None], (_PACK - _NE, 16))], axis=0)

    A_t = _get_sc_norm()(pack)

    Wlin_nm = Wlin.reshape(_HID, _NN, -1).transpose(1, 0, 2).reshape(
        _NN * _HID, -1)

    out = _tc_core(x, Wz, Wh, Wlz, Wlh,
                   bz.reshape(1, _HID), bh.reshape(1, _HID),
                   blz.reshape(1, _HID), blh.reshape(1, _HID),
                   A_t, Wlin_nm, blin.reshape(1, -1))
    return out.reshape(-1)


# speedup vs baseline: 20.2895x; 1.0116x over previous
import functools

import jax
import jax.numpy as jnp
from jax import lax
from jax.experimental import pallas as pl
from jax.experimental.pallas import tpu as pltpu
from jax.experimental.pallas import tpu_sc as plsc

_SEQ = 15
_NN = 3
_NE = 9
_FIN = 512
_HID = 32

_MCOL = _NE
_MROW = _MCOL + _NE * _NN
_PACK = _MROW + _NE * _NN


def _sc_norm_body(pack_hbm, a_hbm, pack_v, a_v):
    c = lax.axis_index("c")
    s = lax.axis_index("s")

    @pl.when((s == 0) & (c == 0))
    def _():
        pltpu.sync_copy(pack_hbm, pack_v)
        ew = [pack_v[e, :] for e in range(_NE)]

        deg = []
        for n in range(_NN):
            d = jnp.full((16,), 1.0, jnp.float32)
            for e in range(_NE):
                d = d + pack_v[_MCOL + _NN * e + n, :] * ew[e]
            deg.append(d)

        dinv = []
        for n in range(_NN):
            x = deg[n]
            y = 1.0 / x
            for _ in range(12):
                y = y * (1.5 - (0.5 * x) * y * y)
            dinv.append(y)

        a = [jnp.zeros((16,), jnp.float32)] * _NE
        for n in range(_NN):
            a[4 * n] = dinv[n] * dinv[n]
        for e in range(_NE):
            mr = [pack_v[_MROW + _NN * e + n, :] for n in range(_NN)]
            mc = [pack_v[_MCOL + _NN * e + n, :] for n in range(_NN)]
            dr = mr[0] * dinv[0] + mr[1] * dinv[1] + mr[2] * dinv[2]
            dc = mc[0] * dinv[0] + mc[1] * dinv[1] + mc[2] * dinv[2]
            norm_e = ew[e] * dr * dc
            for cn in range(_NN):
                t = mc[cn] * norm_e
                for r in range(_NN):
                    a[3 * cn + r] = a[3 * cn + r] + mr[r] * t
        for j in range(_NE):
            a_v[j, :] = a[j]
        pltpu.sync_copy(a_v, a_hbm)


@functools.cache
def _get_sc_norm():
    return functools.partial(
        pl.kernel,
        out_type=jax.ShapeDtypeStruct((_NE, 16), jnp.float32),
        mesh=plsc.VectorSubcoreMesh(core_axis_name="c", subcore_axis_name="s"),
        scratch_types=[
            pltpu.VMEM((_PACK, 16), jnp.float32),
            pltpu.VMEM((_NE, 16), jnp.float32),
        ],
    )(_sc_norm_body)


def _tc_body(xn_ref, wz_ref, wh_ref, wlz_ref, wlh_ref, bz_ref, bh_ref,
             blz_ref, blh_ref, a_ref, wlin_ref, blin_ref, out_ref):
    X = xn_ref[...].transpose(1, 0, 2).reshape(_NN * _SEQ, _FIN)
    W = jnp.concatenate([wz_ref[...], wh_ref[...]], axis=1)
    XW = jnp.dot(X, W, preferred_element_type=jnp.float32)
    A = a_ref[...][:, :_SEQ].T

    parts = []
    for cn in range(_NN):
        acc = (A[:, 3 * cn + 0:3 * cn + 1] * XW[0:15, :]
               + A[:, 3 * cn + 1:3 * cn + 2] * XW[15:30, :]
               + A[:, 3 * cn + 2:3 * cn + 3] * XW[30:45, :])
        parts.append(acc)
    G = jnp.concatenate(parts, axis=0)

    Gz = G[:, 0:_HID] + bz_ref[...]
    Gh = G[:, _HID:2 * _HID] + bh_ref[...]
    Z = jax.nn.sigmoid(
        jnp.dot(Gz, wlz_ref[0:_HID, :], preferred_element_type=jnp.float32)
        + blz_ref[...])
    Ht = jnp.tanh(
        jnp.dot(Gh, wlh_ref[0:_HID, :], preferred_element_type=jnp.float32)
        + blh_ref[...])
    h = jnp.maximum((1.0 - Z) * Ht, 0.0)

    fi = lax.broadcasted_iota(jnp.int32, (_HID, _NN * _HID), 0)
    li = lax.broadcasted_iota(jnp.int32, (_HID, _NN * _HID), 1)
    q = jnp.zeros((1, _NN * _HID), jnp.float32)
    for n in range(_NN):
        pn = jnp.sum(h[_SEQ * n:_SEQ * (n + 1), :], axis=0, keepdims=True) \
            * (1.0 / _SEQ)
        sel = jnp.where(li == _NN * fi + n, 1.0, 0.0)
        q = q + jnp.dot(pn, sel, preferred_element_type=jnp.float32)
    out_ref[...] = jnp.dot(q, wlin_ref[...],
                           preferred_element_type=jnp.float32) + blin_ref[...]


_tc_core = pl.pallas_call(
    _tc_body,
    out_shape=jax.ShapeDtypeStruct((1, 2), jnp.float32),
)


def kernel(x, edge_index, edge_weight, Wz, bz, Wlz, blz, Wr, br, Wlr, blr,
           Wh, bh, Wlh, blh, Wlin, blin):
    row = edge_index[0].astype(jnp.int32)
    col = edge_index[1].astype(jnp.int32)
    n3 = jnp.arange(_NN, dtype=jnp.int32)
    mcol = (col[:, None] == n3).astype(jnp.float32).reshape(-1)
    mrow = (row[:, None] == n3).astype(jnp.float32).reshape(-1)
    masks = jnp.concatenate([mcol, mrow])
    ew16 = jnp.zeros((_NE, 16), jnp.float32).at[:, :_SEQ].set(edge_weight.T)
    pack = jnp.concatenate(
        [ew16, jnp.broadcast_to(masks[:, None], (_PACK - _NE, 16))], axis=0)

    A_t = _get_sc_norm()(pack)

    out = _tc_core(x, Wz, Wh, Wlz, Wlh,
                   bz.reshape(1, _HID), bh.reshape(1, _HID),
                   blz.reshape(1, _HID), blh.reshape(1, _HID),
                   A_t, Wlin, blin.reshape(1, -1))
    return out.reshape(-1)


# speedup vs baseline: 21.5026x; 1.0598x over previous
import functools

import jax
import jax.numpy as jnp
from jax import lax
from jax.experimental import pallas as pl
from jax.experimental.pallas import tpu as pltpu
from jax.experimental.pallas import tpu_sc as plsc

_SEQ = 15
_NN = 3
_NE = 9
_FIN = 512
_HID = 32

_MCOL = _NE
_MROW = _MCOL + _NE * _NN
_PACK = _MROW + _NE * _NN


def _sc_norm_body(pack_hbm, a_hbm, pack_v, a_v):
    pltpu.sync_copy(pack_hbm, pack_v)
    ew = [pack_v[e, :] for e in range(_NE)]

    deg = []
    for n in range(_NN):
        d = jnp.full((16,), 1.0, jnp.float32)
        for e in range(_NE):
            d = d + pack_v[_MCOL + _NN * e + n, :] * ew[e]
        deg.append(d)

    dinv = []
    for n in range(_NN):
        x = deg[n]
        y = 1.0 / x
        for _ in range(12):
            y = y * (1.5 - (0.5 * x) * y * y)
        dinv.append(y)

    a = [jnp.zeros((16,), jnp.float32)] * _NE
    for n in range(_NN):
        a[4 * n] = dinv[n] * dinv[n]
    for e in range(_NE):
        mr = [pack_v[_MROW + _NN * e + n, :] for n in range(_NN)]
        mc = [pack_v[_MCOL + _NN * e + n, :] for n in range(_NN)]
        dr = mr[0] * dinv[0] + mr[1] * dinv[1] + mr[2] * dinv[2]
        dc = mc[0] * dinv[0] + mc[1] * dinv[1] + mc[2] * dinv[2]
        norm_e = ew[e] * dr * dc
        for cn in range(_NN):
            t = mc[cn] * norm_e
            for r in range(_NN):
                a[3 * cn + r] = a[3 * cn + r] + mr[r] * t
    for j in range(_NE):
        a_v[j, :] = a[j]
    pltpu.sync_copy(a_v, a_hbm)


@functools.cache
def _get_sc_norm():
    return functools.partial(
        pl.kernel,
        out_type=jax.ShapeDtypeStruct((_NE, 16), jnp.float32),
        mesh=plsc.VectorSubcoreMesh(core_axis_name="c", subcore_axis_name="s",
                                    num_cores=1, num_subcores=1),
        scratch_types=[
            pltpu.VMEM((_PACK, 16), jnp.float32),
            pltpu.VMEM((_NE, 16), jnp.float32),
        ],
    )(_sc_norm_body)


def _tc_body(xn_ref, wz_ref, wh_ref, wlz_ref, wlh_ref, bz_ref, bh_ref,
             blz_ref, blh_ref, a_ref, wlin_ref, blin_ref, out_ref):
    X = xn_ref[...].transpose(1, 0, 2).reshape(_NN * _SEQ, _FIN)
    W = jnp.concatenate([wz_ref[...], wh_ref[...]], axis=1)
    XW = jnp.dot(X, W, preferred_element_type=jnp.float32)
    A = a_ref[...][:, :_SEQ].T

    parts = []
    for cn in range(_NN):
        acc = (A[:, 3 * cn + 0:3 * cn + 1] * XW[0:15, :]
               + A[:, 3 * cn + 1:3 * cn + 2] * XW[15:30, :]
               + A[:, 3 * cn + 2:3 * cn + 3] * XW[30:45, :])
        parts.append(acc)
    G = jnp.concatenate(parts, axis=0)

    Gz = G[:, 0:_HID] + bz_ref[...]
    Gh = G[:, _HID:2 * _HID] + bh_ref[...]
    Z = jax.nn.sigmoid(
        jnp.dot(Gz, wlz_ref[0:_HID, :], preferred_element_type=jnp.float32)
        + blz_ref[...])
    Ht = jnp.tanh(
        jnp.dot(Gh, wlh_ref[0:_HID, :], preferred_element_type=jnp.float32)
        + blh_ref[...])
    h = jnp.maximum((1.0 - Z) * Ht, 0.0)

    fi = lax.broadcasted_iota(jnp.int32, (_HID, _NN * _HID), 0)
    li = lax.broadcasted_iota(jnp.int32, (_HID, _NN * _HID), 1)
    q = jnp.zeros((1, _NN * _HID), jnp.float32)
    for n in range(_NN):
        pn = jnp.sum(h[_SEQ * n:_SEQ * (n + 1), :], axis=0, keepdims=True) \
            * (1.0 / _SEQ)
        sel = jnp.where(li == _NN * fi + n, 1.0, 0.0)
        q = q + jnp.dot(pn, sel, preferred_element_type=jnp.float32)
    out_ref[...] = jnp.dot(q, wlin_ref[...],
                           preferred_element_type=jnp.float32) + blin_ref[...]


_tc_core = pl.pallas_call(
    _tc_body,
    out_shape=jax.ShapeDtypeStruct((1, 2), jnp.float32),
)


def kernel(x, edge_index, edge_weight, Wz, bz, Wlz, blz, Wr, br, Wlr, blr,
           Wh, bh, Wlh, blh, Wlin, blin):
    row = edge_index[0].astype(jnp.int32)
    col = edge_index[1].astype(jnp.int32)
    n3 = jnp.arange(_NN, dtype=jnp.int32)
    mcol = (col[:, None] == n3).astype(jnp.float32).reshape(-1)
    mrow = (row[:, None] == n3).astype(jnp.float32).reshape(-1)
    masks = jnp.concatenate([mcol, mrow])
    ew16 = jnp.zeros((_NE, 16), jnp.float32).at[:, :_SEQ].set(edge_weight.T)
    pack = jnp.concatenate(
        [ew16, jnp.broadcast_to(masks[:, None], (_PACK - _NE, 16))], axis=0)

    A_t = _get_sc_norm()(pack)

    out = _tc_core(x, Wz, Wh, Wlz, Wlh,
                   bz.reshape(1, _HID), bh.reshape(1, _HID),
                   blz.reshape(1, _HID), blh.reshape(1, _HID),
                   A_t, Wlin, blin.reshape(1, -1))
    return out.reshape(-1)


# speedup vs baseline: 22.1638x; 1.0308x over previous
import functools

import jax
import jax.numpy as jnp
from jax import lax
from jax.experimental import pallas as pl
from jax.experimental.pallas import tpu as pltpu
from jax.experimental.pallas import tpu_sc as plsc

_SEQ = 15
_NN = 3
_NE = 9
_FIN = 512
_HID = 32


def _sc_norm_body(ew_hbm, ei_hbm, a_hbm, ew_v, ei_v, a_v):
    pltpu.sync_copy(ew_hbm, ew_v)
    pltpu.sync_copy(ei_hbm, ei_v)

    rows = [ei_v[0, e] for e in range(_NE)]
    cols = [ei_v[1, e] for e in range(_NE)]
    slots = [3 * cols[e] + rows[e] for e in range(_NE)]

    for t in range(_SEQ):
        for j in range(_NE + _NN):
            a_v[j, t] = jnp.float32(0.0)
        for e in range(_NE):
            w = ew_v[t, e]
            a_v[slots[e], t] = a_v[slots[e], t] + w
            a_v[_NE + cols[e], t] = a_v[_NE + cols[e], t] + w

    for j in range(_NE + _NN):
        a_v[j, _SEQ] = jnp.float32(0.0)
    pltpu.sync_copy(a_v, a_hbm)


@functools.cache
def _get_sc_norm():
    return functools.partial(
        pl.kernel,
        out_type=jax.ShapeDtypeStruct((_NE + _NN, 16), jnp.float32),
        mesh=plsc.ScalarSubcoreMesh(axis_name="c", num_cores=1),
        scratch_types=[
            pltpu.SMEM((_SEQ, _NE), jnp.float32),
            pltpu.SMEM((2, _NE), jnp.int32),
            pltpu.SMEM((_NE + _NN, 16), jnp.float32),
        ],
    )(_sc_norm_body)


def _tc_body(xn_ref, wz_ref, wh_ref, wlz_ref, wlh_ref, bz_ref, bh_ref,
             blz_ref, blh_ref, a_ref, wlin_ref, blin_ref, out_ref):
    X = xn_ref[...].transpose(1, 0, 2).reshape(_NN * _SEQ, _FIN)
    W = jnp.concatenate([wz_ref[...], wh_ref[...]], axis=1)
    XW = jnp.dot(X, W, preferred_element_type=jnp.float32)

    B = a_ref[...]
    dinv = lax.rsqrt(B[_NE:_NE + _NN, :] + 1.0)
    arows = []
    for cn in range(_NN):
        for r in range(_NN):
            ar = dinv[cn:cn + 1, :] * dinv[r:r + 1, :] \
                * B[3 * cn + r:3 * cn + r + 1, :]
            if cn == r:
                ar = ar + dinv[cn:cn + 1, :] * dinv[cn:cn + 1, :]
            arows.append(ar)
    A = jnp.concatenate(arows, axis=0)[:, :_SEQ].T

    parts = []
    for cn in range(_NN):
        acc = (A[:, 3 * cn + 0:3 * cn + 1] * XW[0:15, :]
               + A[:, 3 * cn + 1:3 * cn + 2] * XW[15:30, :]
               + A[:, 3 * cn + 2:3 * cn + 3] * XW[30:45, :])
        parts.append(acc)
    G = jnp.concatenate(parts, axis=0)

    Gz = G[:, 0:_HID] + bz_ref[...]
    Gh = G[:, _HID:2 * _HID] + bh_ref[...]
    Z = jax.nn.sigmoid(
        jnp.dot(Gz, wlz_ref[0:_HID, :], preferred_element_type=jnp.float32)
        + blz_ref[...])
    Ht = jnp.tanh(
        jnp.dot(Gh, wlh_ref[0:_HID, :], preferred_element_type=jnp.float32)
        + blh_ref[...])
    h = jnp.maximum((1.0 - Z) * Ht, 0.0)

    fi = lax.broadcasted_iota(jnp.int32, (_HID, _NN * _HID), 0)
    li = lax.broadcasted_iota(jnp.int32, (_HID, _NN * _HID), 1)
    q = jnp.zeros((1, _NN * _HID), jnp.float32)
    for n in range(_NN):
        pn = jnp.sum(h[_SEQ * n:_SEQ * (n + 1), :], axis=0, keepdims=True) \
            * (1.0 / _SEQ)
        sel = jnp.where(li == _NN * fi + n, 1.0, 0.0)
        q = q + jnp.dot(pn, sel, preferred_element_type=jnp.float32)
    out_ref[...] = jnp.dot(q, wlin_ref[...],
                           preferred_element_type=jnp.float32) + blin_ref[...]


_tc_core = pl.pallas_call(
    _tc_body,
    out_shape=jax.ShapeDtypeStruct((1, 2), jnp.float32),
)


def kernel(x, edge_index, edge_weight, Wz, bz, Wlz, blz, Wr, br, Wlr, blr,
           Wh, bh, Wlh, blh, Wlin, blin):
    A_t = _get_sc_norm()(edge_weight, edge_index.astype(jnp.int32))

    out = _tc_core(x, Wz, Wh, Wlz, Wlh,
                   bz.reshape(1, _HID), bh.reshape(1, _HID),
                   blz.reshape(1, _HID), blh.reshape(1, _HID),
                   A_t, Wlin, blin.reshape(1, -1))
    return out.reshape(-1)


# speedup vs baseline: 22.2158x; 1.0023x over previous
import functools

import jax
import jax.numpy as jnp
from jax import lax
from jax.experimental import pallas as pl
from jax.experimental.pallas import tpu as pltpu
from jax.experimental.pallas import tpu_sc as plsc

_SEQ = 15
_NN = 3
_NE = 9
_FIN = 512
_HID = 32


def _sc_norm_body(ew_hbm, ei_hbm, a_hbm, ew_v, ei_v, a_v):
    pltpu.sync_copy(ew_hbm, ew_v)
    pltpu.sync_copy(ei_hbm, ei_v)

    slots = [3 * ei_v[1, e] + ei_v[0, e] for e in range(_NE)]

    for j in range(_NE):
        for t in range(16):
            a_v[j, t] = jnp.float32(0.0)
    for t in range(_SEQ):
        for e in range(_NE):
            a_v[slots[e], t] = a_v[slots[e], t] + ew_v[t, e]
    pltpu.sync_copy(a_v, a_hbm)


@functools.cache
def _get_sc_norm():
    return functools.partial(
        pl.kernel,
        out_type=jax.ShapeDtypeStruct((_NE, 16), jnp.float32),
        mesh=plsc.ScalarSubcoreMesh(axis_name="c", num_cores=1),
        scratch_types=[
            pltpu.SMEM((_SEQ, _NE), jnp.float32),
            pltpu.SMEM((2, _NE), jnp.int32),
            pltpu.SMEM((_NE, 16), jnp.float32),
        ],
    )(_sc_norm_body)


def _tc_body(xn_ref, wz_ref, wh_ref, wlz_ref, wlh_ref, bz_ref, bh_ref,
             blz_ref, blh_ref, a_ref, wlin_ref, blin_ref, out_ref):
    X = xn_ref[...].transpose(1, 0, 2).reshape(_NN * _SEQ, _FIN)
    W = jnp.concatenate([wz_ref[...], wh_ref[...]], axis=1)
    XW = jnp.dot(X, W, preferred_element_type=jnp.float32)

    B = a_ref[...]
    deg = jnp.concatenate(
        [B[3 * c:3 * c + 1, :] + B[3 * c + 1:3 * c + 2, :]
         + B[3 * c + 2:3 * c + 3, :] for c in range(_NN)], axis=0)
    dinv = lax.rsqrt(deg + 1.0)
    arows = []
    for cn in range(_NN):
        for r in range(_NN):
            ar = dinv[cn:cn + 1, :] * dinv[r:r + 1, :] \
                * B[3 * cn + r:3 * cn + r + 1, :]
            if cn == r:
                ar = ar + dinv[cn:cn + 1, :] * dinv[cn:cn + 1, :]
            arows.append(ar)
    A = jnp.concatenate(arows, axis=0)[:, :_SEQ].T

    parts = []
    for cn in range(_NN):
        acc = (A[:, 3 * cn + 0:3 * cn + 1] * XW[0:15, :]
               + A[:, 3 * cn + 1:3 * cn + 2] * XW[15:30, :]
               + A[:, 3 * cn + 2:3 * cn + 3] * XW[30:45, :])
        parts.append(acc)
    G = jnp.concatenate(parts, axis=0)

    Gz = G[:, 0:_HID] + bz_ref[...]
    Gh = G[:, _HID:2 * _HID] + bh_ref[...]
    Z = jax.nn.sigmoid(
        jnp.dot(Gz, wlz_ref[0:_HID, :], preferred_element_type=jnp.float32)
        + blz_ref[...])
    Ht = jnp.tanh(
        jnp.dot(Gh, wlh_ref[0:_HID, :], preferred_element_type=jnp.float32)
        + blh_ref[...])
    h = jnp.maximum((1.0 - Z) * Ht, 0.0)

    fi = lax.broadcasted_iota(jnp.int32, (_HID, _NN * _HID), 0)
    li = lax.broadcasted_iota(jnp.int32, (_HID, _NN * _HID), 1)
    q = jnp.zeros((1, _NN * _HID), jnp.float32)
    for n in range(_NN):
        pn = jnp.sum(h[_SEQ * n:_SEQ * (n + 1), :], axis=0, keepdims=True) \
            * (1.0 / _SEQ)
        sel = jnp.where(li == _NN * fi + n, 1.0, 0.0)
        q = q + jnp.dot(pn, sel, preferred_element_type=jnp.float32)
    out_ref[...] = jnp.dot(q, wlin_ref[...],
                           preferred_element_type=jnp.float32) + blin_ref[...]


_tc_core = pl.pallas_call(
    _tc_body,
    out_shape=jax.ShapeDtypeStruct((1, 2), jnp.float32),
)


def kernel(x, edge_index, edge_weight, Wz, bz, Wlz, blz, Wr, br, Wlr, blr,
           Wh, bh, Wlh, blh, Wlin, blin):
    A_t = _get_sc_norm()(edge_weight, edge_index.astype(jnp.int32))

    out = _tc_core(x, Wz, Wh, Wlz, Wlh,
                   bz.reshape(1, _HID), bh.reshape(1, _HID),
                   blz.reshape(1, _HID), blh.reshape(1, _HID),
                   A_t, Wlin, blin.reshape(1, -1))
    return out.reshape(-1)
